# baseline (speedup 1.0000x reference)
import jax
import jax.numpy as jnp
from jax.experimental import pallas as pl

_B = 32
_N = 20000
_C = 25
_KC = 5
_PL = 8192


def _decode_block(p_ref, pb_ref, o_ref):
    j = pl.program_id(1)

    @pl.when(j == 0)
    def _math():
        w = pb_ref[2:3, :]
        h = pb_ref[3:4, :]
        o_ref[0] = (p_ref[0] + 1.0) * w
        o_ref[1] = (p_ref[1] + 1.0) * h
        o_ref[2] = jnp.exp(p_ref[2]) * w
        o_ref[3] = jnp.exp(p_ref[3]) * h
        o_ref[4] = p_ref[4]

    @pl.when(j != 0)
    def _copy():
        o_ref[...] = p_ref[...]


def kernel(p, priorbox):
    pt = jnp.transpose(p, (2, 0, 1))
    pbt = jnp.transpose(priorbox, (1, 0))
    out_t = pl.pallas_call(
        _decode_block,
        grid=(pl.cdiv(_N, _PL), _C // _KC),
        in_specs=[
            pl.BlockSpec((_KC, _B, _PL), lambda i, j: (j, 0, i)),
            pl.BlockSpec((4, _PL), lambda i, j: (0, i)),
        ],
        out_specs=pl.BlockSpec((_KC, _B, _PL), lambda i, j: (j, 0, i)),
        out_shape=jax.ShapeDtypeStruct((_C, _B, _N), jnp.float32),
    )(pt, pbt)
    return jnp.transpose(out_t, (1, 2, 0))
